# Initial kernel scaffold; baseline (speedup 1.0000x reference)
import jax
import jax.numpy as jnp
from jax.experimental import pallas as pl


def kernel(embeddings):
    raise NotImplementedError("write your pallas kernel here")


# speedup vs baseline: 4.9460x; 4.9460x over previous
import jax
import jax.numpy as jnp
from jax.experimental import pallas as pl

_K = 20
_ROWS = 256
_NEG = -3.0


def _norm_body(emb_ref, out_ref):
    x = emb_ref[...]
    sq = jnp.sum(x * x, axis=1, keepdims=True)
    out_ref[...] = x * jax.lax.rsqrt(jnp.maximum(sq, 1e-12))


def _topk_body(lhs_ref, rhs_ref, vals_ref, idxs_ref):
    i = pl.program_id(0)
    a = lhs_ref[...]
    b = rhs_ref[...]
    n = b.shape[0]
    sim = jax.lax.dot_general(
        a, b, (((1,), (1,)), ((), ())), preferred_element_type=jnp.float32
    )
    col = jax.lax.broadcasted_iota(jnp.int32, sim.shape, 1)
    row = i * _ROWS + jax.lax.broadcasted_iota(jnp.int32, (sim.shape[0], 1), 0)
    vals = []
    idxs = []
    for _ in range(_K):
        m = jnp.max(sim, axis=1, keepdims=True)
        idx = jnp.min(jnp.where(sim >= m, col, n), axis=1, keepdims=True)
        sim = jnp.where(col == idx, _NEG, sim)
        vals.append(jnp.where(idx == row, 0.0, m))
        idxs.append(idx)
    vals_ref[...] = jnp.concatenate(vals, axis=1)
    idxs_ref[...] = jnp.concatenate(idxs, axis=1)


def kernel(embeddings):
    n, d = embeddings.shape
    norm = pl.pallas_call(
        _norm_body,
        out_shape=jax.ShapeDtypeStruct((n, d), jnp.float32),
    )(embeddings)

    grid = (pl.cdiv(n, _ROWS),)
    vals, idxs = pl.pallas_call(
        _topk_body,
        grid=grid,
        in_specs=[
            pl.BlockSpec((_ROWS, d), lambda i: (i, 0)),
            pl.BlockSpec((n, d), lambda i: (0, 0)),
        ],
        out_specs=[
            pl.BlockSpec((_ROWS, _K), lambda i: (i, 0)),
            pl.BlockSpec((_ROWS, _K), lambda i: (i, 0)),
        ],
        out_shape=[
            jax.ShapeDtypeStruct((n, _K), jnp.float32),
            jax.ShapeDtypeStruct((n, _K), jnp.int32),
        ],
    )(norm, norm)
    return vals, idxs


# speedup vs baseline: 4.9496x; 1.0007x over previous
import jax
import jax.numpy as jnp
from jax.experimental import pallas as pl

_K = 20
_ROWS = 256
_NEG = -3.0


def _norm_body(emb_ref, out_ref):
    x = emb_ref[...]
    sq = jnp.sum(x * x, axis=1, keepdims=True)
    out_ref[...] = x * jax.lax.rsqrt(jnp.maximum(sq, 1e-12))


def _topk_body(lhs_ref, rhs_ref, vals_ref, idxs_ref):
    i = pl.program_id(0)
    a = lhs_ref[...]
    b = rhs_ref[...]
    n = b.shape[0]
    sim = jax.lax.dot_general(
        a, b, (((1,), (1,)), ((), ())), preferred_element_type=jnp.float32
    )
    col = jax.lax.broadcasted_iota(jnp.int32, sim.shape, 1)
    row = i * _ROWS + jax.lax.broadcasted_iota(jnp.int32, (sim.shape[0], 1), 0)
    vals = []
    idxs = []
    for _ in range(_K):
        m = jnp.max(sim, axis=1, keepdims=True)
        idx = jnp.argmax(sim, axis=1).astype(jnp.int32)[:, None]
        sim = jnp.where(col == idx, _NEG, sim)
        vals.append(jnp.where(idx == row, 0.0, m))
        idxs.append(idx)
    vals_ref[...] = jnp.concatenate(vals, axis=1)
    idxs_ref[...] = jnp.concatenate(idxs, axis=1)


def kernel(embeddings):
    n, d = embeddings.shape
    norm = pl.pallas_call(
        _norm_body,
        out_shape=jax.ShapeDtypeStruct((n, d), jnp.float32),
    )(embeddings)

    grid = (pl.cdiv(n, _ROWS),)
    vals, idxs = pl.pallas_call(
        _topk_body,
        grid=grid,
        in_specs=[
            pl.BlockSpec((_ROWS, d), lambda i: (i, 0)),
            pl.BlockSpec((n, d), lambda i: (0, 0)),
        ],
        out_specs=[
            pl.BlockSpec((_ROWS, _K), lambda i: (i, 0)),
            pl.BlockSpec((_ROWS, _K), lambda i: (i, 0)),
        ],
        out_shape=[
            jax.ShapeDtypeStruct((n, _K), jnp.float32),
            jax.ShapeDtypeStruct((n, _K), jnp.int32),
        ],
    )(norm, norm)
    return vals, idxs
